# baseline (speedup 1.0000x reference)
import jax
import jax.numpy as jnp
from jax import lax
from jax.experimental import pallas as pl
from jax.experimental.pallas import tpu as pltpu
from jax.experimental.pallas import tpu_sc as plsc

F32 = jnp.float32

NC = 2
NS = 16
NW = NC * NS

N = 10000
E = 320000
D = 128
DE = 16
H = 64

EPW = E // NW
CB = 128
NCH = EPW // CB
REM = EPW - NCH * CB
PAD = (NCH + 1) * CB

def _z16i():
    return jnp.zeros((16,), jnp.int32)


def _z16f():
    return jnp.zeros((16,), F32)


def _sc_gather_body(ps_hbm, pd_hbm, row_hbm, col_hbm, g_hbm, rowb, colb, abuf0, abuf1,
                    bbuf0, bbuf1, cbuf0, cbuf1, gsem0, gsem1, wsem0, wsem1):
    wid = lax.axis_index("s") * NC + lax.axis_index("c")
    base = wid * EPW

    for k in range(7):
        rowb[pl.ds(EPW + 16 * k, 16)] = _z16i()
        colb[pl.ds(EPW + 16 * k, 16)] = _z16i()
    pltpu.sync_copy(row_hbm.at[pl.ds(base, EPW)], rowb.at[pl.ds(0, EPW)])
    pltpu.sync_copy(col_hbm.at[pl.ds(base, EPW)], colb.at[pl.ds(0, EPW)])

    abufs, bbufs, cbufs = (abuf0, abuf1), (bbuf0, bbuf1), (cbuf0, cbuf1)
    gsems, wsems = (gsem0, gsem1), (wsem0, wsem1)

    def issue(t, p):
        start = t * CB
        pltpu.async_copy(ps_hbm.at[rowb.at[pl.ds(start, CB)]], abufs[p],
                         gsems[p])
        pltpu.async_copy(pd_hbm.at[colb.at[pl.ds(start, CB)]], bbufs[p],
                         gsems[p])

    def wait_gather(p):
        pltpu.make_async_copy(ps_hbm.at[rowb.at[pl.ds(0, CB)]], abufs[p],
                              gsems[p]).wait()
        pltpu.make_async_copy(pd_hbm.at[colb.at[pl.ds(0, CB)]], bbufs[p],
                              gsems[p]).wait()

    def wait_write(p):
        pltpu.make_async_copy(cbufs[p], g_hbm.at[pl.ds(0, CB)],
                              wsems[p]).wait()

    def combine(t, p):
        def addrow(i, carry):
            for r in range(4):
                for k in range(4):
                    cbufs[p][4 * i + r, pl.ds(k * 16, 16)] = (
                        abufs[p][4 * i + r, pl.ds(k * 16, 16)]
                        + bbufs[p][4 * i + r, pl.ds(k * 16, 16)])
            return carry

        lax.fori_loop(0, CB // 4, addrow, 0)
        pltpu.async_copy(cbufs[p], g_hbm.at[pl.ds(base + t * CB, CB)],
                         wsems[p])

    issue(0, 0)

    def pair(i, carry):
        for p in range(2):
            t = 2 * i + p

            @pl.when(t + 1 < NCH)
            def _():
                issue(t + 1, 1 - p)

            wait_gather(p)

            @pl.when(i >= 1)
            def _():
                wait_write(p)

            combine(t, p)

        return carry

    lax.fori_loop(0, NCH // 2, pair, 0)
    wait_write(0)
    wait_write(1)

    start = NCH * CB
    ca = pltpu.async_copy(ps_hbm.at[rowb.at[pl.ds(start, CB)]], abufs[0],
                          gsems[0])
    cb = pltpu.async_copy(pd_hbm.at[colb.at[pl.ds(start, CB)]], bbufs[0],
                          gsems[0])
    ca.wait()
    cb.wait()

    def addrow(i, carry):
        for k in range(4):
            cbufs[0][i, pl.ds(k * 16, 16)] = (
                abufs[0][i, pl.ds(k * 16, 16)]
                + bbufs[0][i, pl.ds(k * 16, 16)])
        return carry

    lax.fori_loop(0, REM, addrow, 0)
    pltpu.sync_copy(cbufs[0].at[pl.ds(0, REM)],
                    g_hbm.at[pl.ds(base + start, REM)])


def _sc_gather(ps, pd, row, col):
    run = pl.kernel(
        _sc_gather_body,
        out_type=jax.ShapeDtypeStruct((E, H), F32),
        mesh=plsc.VectorSubcoreMesh(core_axis_name="c", subcore_axis_name="s"),
        scratch_types=[
            pltpu.VMEM((PAD,), jnp.int32),
            pltpu.VMEM((PAD,), jnp.int32),
            pltpu.VMEM((CB, H), F32),
            pltpu.VMEM((CB, H), F32),
            pltpu.VMEM((CB, H), F32),
            pltpu.VMEM((CB, H), F32),
            pltpu.VMEM((CB, H), F32),
            pltpu.VMEM((CB, H), F32),
            pltpu.SemaphoreType.DMA,
            pltpu.SemaphoreType.DMA,
            pltpu.SemaphoreType.DMA,
            pltpu.SemaphoreType.DMA,
        ],
        compiler_params=pltpu.CompilerParams(use_tc_tiling_on_sc=False),
    )
    return run(ps, pd, row, col)


_NH = N // 2


_CBS = 128
_NCHUNKS_S = E // _CBS
_TRIPS = 80


def _sc_scatter_body(eo_hbm, col_hbm, out_hbm, colb0, colb1, ebuf0, ebuf1,
                     accum1d, sem0, sem1):
    cid = lax.axis_index("c")
    sid = lax.axis_index("s")
    wid = sid * NC + cid
    colbs, ebufs, sems = (colb0, colb1), (ebuf0, ebuf1), (sem0, sem1)

    def chunk_of(t):
        return jnp.minimum(wid + NW * t, _NCHUNKS_S - 1)

    def issue(t, p):
        c = chunk_of(t)
        pltpu.async_copy(col_hbm.at[pl.ds(c * _CBS, _CBS)],
                         colbs[p].at[pl.ds(0, _CBS)], sems[p])
        pltpu.async_copy(eo_hbm.at[pl.ds(c * _CBS, _CBS)], ebufs[p], sems[p])

    def wait(p):
        pltpu.make_async_copy(col_hbm.at[pl.ds(0, _CBS)],
                              colbs[p].at[pl.ds(0, _CBS)], sems[p]).wait()
        pltpu.make_async_copy(eo_hbm.at[pl.ds(0, _CBS)], ebufs[p],
                              sems[p]).wait()

    for h in range(2):
        node_base = h * _NH

        def zrow(i, carry):
            accum1d[pl.ds(i * 16, 16)] = _z16f()
            return carry

        lax.fori_loop(0, _NH + 1, zrow, 0)

        issue(0, 0)

        def pair(i, carry):
            for p in range(2):
                t = 2 * i + p

                @pl.when(t + 1 < _TRIPS)
                def _():
                    issue(t + 1, 1 - p)

                wait(p)

                @pl.when(wid + NW * t < _NCHUNKS_S)
                def _():
                    colb, ebuf = colbs[p], ebufs[p]

                    def group(g, carry2):
                        relv = colb[pl.ds(g * 16, 16)] - node_base
                        okv = jnp.logical_and(relv >= 0, relv < _NH)
                        offv = jnp.where(okv, relv, _NH) * DE
                        for l in range(16):
                            s16 = pl.ds(offv[l], 16)
                            accum1d[s16] = (accum1d[s16]
                                            + ebuf[g * 16 + l, pl.ds(0, DE)])
                        return carry2

                    lax.fori_loop(0, _CBS // 16, group, 0)

            return carry

        lax.fori_loop(0, _TRIPS // 2, pair, 0)
        pltpu.sync_copy(accum1d.at[pl.ds(0, _NH * DE)], out_hbm.at[wid, h])


def _sc_scatter(e_out, col):
    run = pl.kernel(
        _sc_scatter_body,
        out_type=jax.ShapeDtypeStruct((NW, 2, _NH * DE), F32),
        mesh=plsc.VectorSubcoreMesh(core_axis_name="c", subcore_axis_name="s"),
        scratch_types=[
            pltpu.VMEM((_CBS + 16,), jnp.int32),
            pltpu.VMEM((_CBS + 16,), jnp.int32),
            pltpu.VMEM((_CBS, DE), F32),
            pltpu.VMEM((_CBS, DE), F32),
            pltpu.VMEM(((_NH + 1) * DE,), F32),
            pltpu.SemaphoreType.DMA,
            pltpu.SemaphoreType.DMA,
        ],
        compiler_params=pltpu.CompilerParams(use_tc_tiling_on_sc=False),
    )
    return run(e_out, col)


def _tc_precompute(x, w_cat):
    def body(x_ref, w_ref, ps_ref, pd_ref):
        p = jnp.dot(x_ref[...], w_ref[...], preferred_element_type=F32)
        ps_ref[...] = p[:, :H]
        pd_ref[...] = p[:, H:]

    nb = 10
    rb = N // nb
    return pl.pallas_call(
        body,
        grid=(nb,),
        in_specs=[
            pl.BlockSpec((rb, D), lambda i: (i, 0)),
            pl.BlockSpec((D, 2 * H), lambda i: (0, 0)),
        ],
        out_specs=[
            pl.BlockSpec((rb, H), lambda i: (i, 0)),
            pl.BlockSpec((rb, H), lambda i: (i, 0)),
        ],
        out_shape=[
            jax.ShapeDtypeStruct((N, H), F32),
            jax.ShapeDtypeStruct((N, H), F32),
        ],
    )(x, w_cat)


def _tc_edge_mlp(g, edge_attr, w1e, b1, w2, b2):
    def body(g_ref, ea_ref, w1_ref, b1_ref, w2_ref, b2_ref, eo_ref):
        ea = ea_ref[...]
        h = jnp.maximum(
            g_ref[...] + jnp.dot(ea, w1_ref[...], preferred_element_type=F32)
            + b1_ref[...], 0.0)
        eo_ref[...] = (jnp.dot(h, w2_ref[...], preferred_element_type=F32)
                       + b2_ref[...] + ea)

    nb = 80
    rb = E // nb
    return pl.pallas_call(
        body,
        grid=(nb,),
        in_specs=[
            pl.BlockSpec((rb, H), lambda i: (i, 0)),
            pl.BlockSpec((rb, DE), lambda i: (i, 0)),
            pl.BlockSpec((DE, H), lambda i: (0, 0)),
            pl.BlockSpec((1, H), lambda i: (0, 0)),
            pl.BlockSpec((H, DE), lambda i: (0, 0)),
            pl.BlockSpec((1, DE), lambda i: (0, 0)),
        ],
        out_specs=pl.BlockSpec((rb, DE), lambda i: (i, 0)),
        out_shape=jax.ShapeDtypeStruct((E, DE), F32),
    )(g, edge_attr, w1e, b1, w2, b2)


def _tc_node_mlp(x, parts, u, wn_x, wn_a, wn_u, bn1, wn2, bn2):
    def body(x_ref, p_ref, u_ref, wx_ref, wa_ref, wu_ref, b1_ref, w2_ref,
             b2_ref, xo_ref):
        xv = x_ref[...]
        agg = jnp.sum(p_ref[...], axis=0)
        pre = (jnp.dot(xv, wx_ref[...], preferred_element_type=F32)
               + jnp.dot(agg, wa_ref[...], preferred_element_type=F32)
               + jnp.dot(u_ref[...], wu_ref[...], preferred_element_type=F32)
               + b1_ref[...])
        hn = jnp.maximum(pre, 0.0)
        xo_ref[...] = (jnp.dot(hn, w2_ref[...], preferred_element_type=F32)
                       + b2_ref[...] + xv)

    nb = 10
    rb = N // nb
    return pl.pallas_call(
        body,
        grid=(nb,),
        in_specs=[
            pl.BlockSpec((rb, D), lambda i: (i, 0)),
            pl.BlockSpec((NW, rb, DE), lambda i: (0, i, 0)),
            pl.BlockSpec((1, DG), lambda i: (0, 0)),
            pl.BlockSpec((D, H), lambda i: (0, 0)),
            pl.BlockSpec((DE, H), lambda i: (0, 0)),
            pl.BlockSpec((DG, H), lambda i: (0, 0)),
            pl.BlockSpec((1, H), lambda i: (0, 0)),
            pl.BlockSpec((H, D), lambda i: (0, 0)),
            pl.BlockSpec((1, D), lambda i: (0, 0)),
        ],
        out_specs=pl.BlockSpec((rb, D), lambda i: (i, 0)),
        out_shape=jax.ShapeDtypeStruct((N, D), F32),
    )(x, parts, u, wn_x, wn_a, wn_u, bn1, wn2, bn2)


DG = 16


def kernel(x, edge_index, edge_attr, u, batch, We1, be1, We2, be2,
           Wn1, bn1, Wn2, bn2):
    w1e = We1[2 * D:, :]
    wn_x = Wn1[:D, :]
    wn_a = Wn1[D:D + DE, :]
    wn_u = Wn1[D + DE:, :]

    w_cat = jnp.concatenate([We1[:D, :], We1[D:2 * D, :]], axis=1)

    row = edge_index[0]
    col = edge_index[1]
    ps, pd = _tc_precompute(x, w_cat)
    g = _sc_gather(ps, pd, row, col)
    e_out = _tc_edge_mlp(g, edge_attr, w1e, be1.reshape(1, H),
                         We2, be2.reshape(1, DE))
    parts = _sc_scatter(e_out, col).reshape(NW, N, DE)
    x_out = _tc_node_mlp(x, parts, u, wn_x, wn_a, wn_u,
                         bn1.reshape(1, H), Wn2, bn2.reshape(1, D))
    return (x_out, e_out, u)


# speedup vs baseline: 1.1030x; 1.1030x over previous
import jax
import jax.numpy as jnp
from jax import lax
from jax.experimental import pallas as pl
from jax.experimental.pallas import tpu as pltpu
from jax.experimental.pallas import tpu_sc as plsc

F32 = jnp.float32

NC = 2
NS = 16
NW = NC * NS

N = 10000
E = 320000
D = 128
DE = 16
H = 64

EPW = E // NW
CB = 128
NCH = EPW // CB
REM = EPW - NCH * CB
PAD = (NCH + 1) * CB

def _z16i():
    return jnp.zeros((16,), jnp.int32)


def _z16f():
    return jnp.zeros((16,), F32)


def _sc_gather_body(ps_hbm, pd_hbm, row_hbm, col_hbm, g_hbm, rowb, colb, abuf0, abuf1,
                    bbuf0, bbuf1, cbuf0, cbuf1, gsem0, gsem1, wsem0, wsem1):
    wid = lax.axis_index("s") * NC + lax.axis_index("c")
    base = wid * EPW

    for k in range(7):
        rowb[pl.ds(EPW + 16 * k, 16)] = _z16i()
        colb[pl.ds(EPW + 16 * k, 16)] = _z16i()
    pltpu.sync_copy(row_hbm.at[pl.ds(base, EPW)], rowb.at[pl.ds(0, EPW)])
    pltpu.sync_copy(col_hbm.at[pl.ds(base, EPW)], colb.at[pl.ds(0, EPW)])

    abufs, bbufs, cbufs = (abuf0, abuf1), (bbuf0, bbuf1), (cbuf0, cbuf1)
    gsems, wsems = (gsem0, gsem1), (wsem0, wsem1)

    def issue(t, p):
        start = t * CB
        pltpu.async_copy(ps_hbm.at[rowb.at[pl.ds(start, CB)]], abufs[p],
                         gsems[p])
        pltpu.async_copy(pd_hbm.at[colb.at[pl.ds(start, CB)]], bbufs[p],
                         gsems[p])

    def wait_gather(p):
        pltpu.make_async_copy(ps_hbm.at[rowb.at[pl.ds(0, CB)]], abufs[p],
                              gsems[p]).wait()
        pltpu.make_async_copy(pd_hbm.at[colb.at[pl.ds(0, CB)]], bbufs[p],
                              gsems[p]).wait()

    def wait_write(p):
        pltpu.make_async_copy(cbufs[p], g_hbm.at[pl.ds(0, CB)],
                              wsems[p]).wait()

    def combine(t, p):
        def addrow(i, carry):
            for r in range(4):
                for k in range(4):
                    cbufs[p][4 * i + r, pl.ds(k * 16, 16)] = (
                        abufs[p][4 * i + r, pl.ds(k * 16, 16)]
                        + bbufs[p][4 * i + r, pl.ds(k * 16, 16)])
            return carry

        lax.fori_loop(0, CB // 4, addrow, 0)
        pltpu.async_copy(cbufs[p], g_hbm.at[pl.ds(base + t * CB, CB)],
                         wsems[p])

    issue(0, 0)

    def pair(i, carry):
        for p in range(2):
            t = 2 * i + p

            @pl.when(t + 1 < NCH)
            def _():
                issue(t + 1, 1 - p)

            wait_gather(p)

            @pl.when(i >= 1)
            def _():
                wait_write(p)

            combine(t, p)

        return carry

    lax.fori_loop(0, NCH // 2, pair, 0)
    wait_write(0)
    wait_write(1)

    start = NCH * CB
    ca = pltpu.async_copy(ps_hbm.at[rowb.at[pl.ds(start, CB)]], abufs[0],
                          gsems[0])
    cb = pltpu.async_copy(pd_hbm.at[colb.at[pl.ds(start, CB)]], bbufs[0],
                          gsems[0])
    ca.wait()
    cb.wait()

    def addrow(i, carry):
        for k in range(4):
            cbufs[0][i, pl.ds(k * 16, 16)] = (
                abufs[0][i, pl.ds(k * 16, 16)]
                + bbufs[0][i, pl.ds(k * 16, 16)])
        return carry

    lax.fori_loop(0, REM, addrow, 0)
    pltpu.sync_copy(cbufs[0].at[pl.ds(0, REM)],
                    g_hbm.at[pl.ds(base + start, REM)])


def _sc_gather(ps, pd, row, col):
    run = pl.kernel(
        _sc_gather_body,
        out_type=jax.ShapeDtypeStruct((E, H), F32),
        mesh=plsc.VectorSubcoreMesh(core_axis_name="c", subcore_axis_name="s"),
        scratch_types=[
            pltpu.VMEM((PAD,), jnp.int32),
            pltpu.VMEM((PAD,), jnp.int32),
            pltpu.VMEM((CB, H), F32),
            pltpu.VMEM((CB, H), F32),
            pltpu.VMEM((CB, H), F32),
            pltpu.VMEM((CB, H), F32),
            pltpu.VMEM((CB, H), F32),
            pltpu.VMEM((CB, H), F32),
            pltpu.SemaphoreType.DMA,
            pltpu.SemaphoreType.DMA,
            pltpu.SemaphoreType.DMA,
            pltpu.SemaphoreType.DMA,
        ],
        compiler_params=pltpu.CompilerParams(use_tc_tiling_on_sc=False),
    )
    return run(ps, pd, row, col)


_NH = N // 2


_CBS = 128
_NCHUNKS_S = E // _CBS
_TRIPS = 80


def _sc_scatter_body(eo_hbm, col_hbm, out_hbm, colb0, colb1, ebuf0, ebuf1,
                     accum1d, sem0, sem1):
    cid = lax.axis_index("c")
    sid = lax.axis_index("s")
    wid = sid * NC + cid
    colbs, ebufs, sems = (colb0, colb1), (ebuf0, ebuf1), (sem0, sem1)

    def chunk_of(t):
        return jnp.minimum(wid + NW * t, _NCHUNKS_S - 1)

    def issue(t, p):
        c = chunk_of(t)
        pltpu.async_copy(col_hbm.at[pl.ds(c * _CBS, _CBS)],
                         colbs[p].at[pl.ds(0, _CBS)], sems[p])
        pltpu.async_copy(eo_hbm.at[pl.ds(c * _CBS, _CBS)], ebufs[p], sems[p])

    def wait(p):
        pltpu.make_async_copy(col_hbm.at[pl.ds(0, _CBS)],
                              colbs[p].at[pl.ds(0, _CBS)], sems[p]).wait()
        pltpu.make_async_copy(eo_hbm.at[pl.ds(0, _CBS)], ebufs[p],
                              sems[p]).wait()

    for h in range(2):
        node_base = h * _NH

        def zrow(i, carry):
            accum1d[pl.ds(i * 16, 16)] = _z16f()
            return carry

        lax.fori_loop(0, _NH + 1, zrow, 0)

        issue(0, 0)

        def pair(i, carry):
            for p in range(2):
                t = 2 * i + p

                @pl.when(t + 1 < _TRIPS)
                def _():
                    issue(t + 1, 1 - p)

                wait(p)

                @pl.when(wid + NW * t < _NCHUNKS_S)
                def _():
                    colb, ebuf = colbs[p], ebufs[p]

                    def group(g, carry2):
                        relv = colb[pl.ds(g * 16, 16)] - node_base
                        okv = jnp.logical_and(relv >= 0, relv < _NH)
                        offv = jnp.where(okv, relv, _NH) * DE
                        for l in range(16):
                            s16 = pl.ds(offv[l], 16)
                            accum1d[s16] = (accum1d[s16]
                                            + ebuf[g * 16 + l, pl.ds(0, DE)])
                        return carry2

                    lax.fori_loop(0, _CBS // 16, group, 0)

            return carry

        lax.fori_loop(0, _TRIPS // 2, pair, 0)
        pltpu.sync_copy(accum1d.at[pl.ds(0, _NH * DE)], out_hbm.at[wid, h])


def _sc_scatter(e_out, col):
    run = pl.kernel(
        _sc_scatter_body,
        out_type=jax.ShapeDtypeStruct((NW, 2, _NH * DE), F32),
        mesh=plsc.VectorSubcoreMesh(core_axis_name="c", subcore_axis_name="s"),
        scratch_types=[
            pltpu.VMEM((_CBS + 16,), jnp.int32),
            pltpu.VMEM((_CBS + 16,), jnp.int32),
            pltpu.VMEM((_CBS, DE), F32),
            pltpu.VMEM((_CBS, DE), F32),
            pltpu.VMEM(((_NH + 1) * DE,), F32),
            pltpu.SemaphoreType.DMA,
            pltpu.SemaphoreType.DMA,
        ],
    )
    return run(e_out, col)


def _tc_precompute(x, w_cat):
    def body(x_ref, w_ref, ps_ref, pd_ref):
        p = jnp.dot(x_ref[...], w_ref[...], preferred_element_type=F32)
        ps_ref[...] = p[:, :H]
        pd_ref[...] = p[:, H:]

    nb = 10
    rb = N // nb
    return pl.pallas_call(
        body,
        grid=(nb,),
        in_specs=[
            pl.BlockSpec((rb, D), lambda i: (i, 0)),
            pl.BlockSpec((D, 2 * H), lambda i: (0, 0)),
        ],
        out_specs=[
            pl.BlockSpec((rb, H), lambda i: (i, 0)),
            pl.BlockSpec((rb, H), lambda i: (i, 0)),
        ],
        out_shape=[
            jax.ShapeDtypeStruct((N, H), F32),
            jax.ShapeDtypeStruct((N, H), F32),
        ],
    )(x, w_cat)


def _tc_edge_mlp(g, edge_attr, w1e, b1, w2, b2):
    def body(g_ref, ea_ref, w1_ref, b1_ref, w2_ref, b2_ref, eo_ref):
        ea = ea_ref[...]
        h = jnp.maximum(
            g_ref[...] + jnp.dot(ea, w1_ref[...], preferred_element_type=F32)
            + b1_ref[...], 0.0)
        eo_ref[...] = (jnp.dot(h, w2_ref[...], preferred_element_type=F32)
                       + b2_ref[...] + ea)

    nb = 80
    rb = E // nb
    return pl.pallas_call(
        body,
        grid=(nb,),
        in_specs=[
            pl.BlockSpec((rb, H), lambda i: (i, 0)),
            pl.BlockSpec((rb, DE), lambda i: (i, 0)),
            pl.BlockSpec((DE, H), lambda i: (0, 0)),
            pl.BlockSpec((1, H), lambda i: (0, 0)),
            pl.BlockSpec((H, DE), lambda i: (0, 0)),
            pl.BlockSpec((1, DE), lambda i: (0, 0)),
        ],
        out_specs=pl.BlockSpec((rb, DE), lambda i: (i, 0)),
        out_shape=jax.ShapeDtypeStruct((E, DE), F32),
    )(g, edge_attr, w1e, b1, w2, b2)


def _tc_node_mlp(x, parts, u, wn_x, wn_a, wn_u, bn1, wn2, bn2):
    def body(x_ref, p_ref, u_ref, wx_ref, wa_ref, wu_ref, b1_ref, w2_ref,
             b2_ref, xo_ref):
        xv = x_ref[...]
        agg = jnp.sum(p_ref[...], axis=0)
        pre = (jnp.dot(xv, wx_ref[...], preferred_element_type=F32)
               + jnp.dot(agg, wa_ref[...], preferred_element_type=F32)
               + jnp.dot(u_ref[...], wu_ref[...], preferred_element_type=F32)
               + b1_ref[...])
        hn = jnp.maximum(pre, 0.0)
        xo_ref[...] = (jnp.dot(hn, w2_ref[...], preferred_element_type=F32)
                       + b2_ref[...] + xv)

    nb = 10
    rb = N // nb
    return pl.pallas_call(
        body,
        grid=(nb,),
        in_specs=[
            pl.BlockSpec((rb, D), lambda i: (i, 0)),
            pl.BlockSpec((NW, rb, DE), lambda i: (0, i, 0)),
            pl.BlockSpec((1, DG), lambda i: (0, 0)),
            pl.BlockSpec((D, H), lambda i: (0, 0)),
            pl.BlockSpec((DE, H), lambda i: (0, 0)),
            pl.BlockSpec((DG, H), lambda i: (0, 0)),
            pl.BlockSpec((1, H), lambda i: (0, 0)),
            pl.BlockSpec((H, D), lambda i: (0, 0)),
            pl.BlockSpec((1, D), lambda i: (0, 0)),
        ],
        out_specs=pl.BlockSpec((rb, D), lambda i: (i, 0)),
        out_shape=jax.ShapeDtypeStruct((N, D), F32),
    )(x, parts, u, wn_x, wn_a, wn_u, bn1, wn2, bn2)


DG = 16


def kernel(x, edge_index, edge_attr, u, batch, We1, be1, We2, be2,
           Wn1, bn1, Wn2, bn2):
    w1e = We1[2 * D:, :]
    wn_x = Wn1[:D, :]
    wn_a = Wn1[D:D + DE, :]
    wn_u = Wn1[D + DE:, :]

    w_cat = jnp.concatenate([We1[:D, :], We1[D:2 * D, :]], axis=1)

    row = edge_index[0]
    col = edge_index[1]
    ps, pd = _tc_precompute(x, w_cat)
    g = _sc_gather(ps, pd, row, col)
    e_out = _tc_edge_mlp(g, edge_attr, w1e, be1.reshape(1, H),
                         We2, be2.reshape(1, DE))
    parts = _sc_scatter(e_out, col).reshape(NW, N, DE)
    x_out = _tc_node_mlp(x, parts, u, wn_x, wn_a, wn_u,
                         bn1.reshape(1, H), Wn2, bn2.reshape(1, D))
    return (x_out, e_out, u)


# speedup vs baseline: 1.1040x; 1.0009x over previous
import jax
import jax.numpy as jnp
from jax import lax
from jax.experimental import pallas as pl
from jax.experimental.pallas import tpu as pltpu
from jax.experimental.pallas import tpu_sc as plsc

F32 = jnp.float32

NC = 2
NS = 16
NW = NC * NS

N = 10000
E = 320000
D = 128
DE = 16
H = 64

EPW = E // NW
CB = 128
NCH = EPW // CB
REM = EPW - NCH * CB
PAD = (NCH + 1) * CB

def _z16i():
    return jnp.zeros((16,), jnp.int32)


def _z16f():
    return jnp.zeros((16,), F32)


def _sc_gather_body(ps_hbm, pd_hbm, row_hbm, col_hbm, g_hbm, rowb, colb,
                    abuf0, abuf1, abuf2, bbuf0, bbuf1, bbuf2,
                    cbuf0, cbuf1, cbuf2, gsem0, gsem1, gsem2,
                    wsem0, wsem1, wsem2):
    wid = lax.axis_index("s") * NC + lax.axis_index("c")
    base = wid * EPW

    for k in range(7):
        rowb[pl.ds(EPW + 16 * k, 16)] = _z16i()
        colb[pl.ds(EPW + 16 * k, 16)] = _z16i()
    pltpu.sync_copy(row_hbm.at[pl.ds(base, EPW)], rowb.at[pl.ds(0, EPW)])
    pltpu.sync_copy(col_hbm.at[pl.ds(base, EPW)], colb.at[pl.ds(0, EPW)])

    abufs = (abuf0, abuf1, abuf2)
    bbufs = (bbuf0, bbuf1, bbuf2)
    cbufs = (cbuf0, cbuf1, cbuf2)
    gsems = (gsem0, gsem1, gsem2)
    wsems = (wsem0, wsem1, wsem2)

    def issue(t, p):
        start = t * CB
        pltpu.async_copy(ps_hbm.at[rowb.at[pl.ds(start, CB)]], abufs[p],
                         gsems[p])
        pltpu.async_copy(pd_hbm.at[colb.at[pl.ds(start, CB)]], bbufs[p],
                         gsems[p])

    def wait_gather(p):
        pltpu.make_async_copy(ps_hbm.at[rowb.at[pl.ds(0, CB)]], abufs[p],
                              gsems[p]).wait()
        pltpu.make_async_copy(pd_hbm.at[colb.at[pl.ds(0, CB)]], bbufs[p],
                              gsems[p]).wait()

    def wait_write(p):
        pltpu.make_async_copy(cbufs[p], g_hbm.at[pl.ds(0, CB)],
                              wsems[p]).wait()

    def combine(t, p):
        def addrow(i, carry):
            for r in range(4):
                for k in range(4):
                    cbufs[p][4 * i + r, pl.ds(k * 16, 16)] = (
                        abufs[p][4 * i + r, pl.ds(k * 16, 16)]
                        + bbufs[p][4 * i + r, pl.ds(k * 16, 16)])
            return carry

        lax.fori_loop(0, CB // 4, addrow, 0)
        pltpu.async_copy(cbufs[p], g_hbm.at[pl.ds(base + t * CB, CB)],
                         wsems[p])

    issue(0, 0)
    issue(1, 1)

    def triple(i, carry):
        for p in range(3):
            t = 3 * i + p

            @pl.when(t + 2 < NCH)
            def _():
                issue(t + 2, (p + 2) % 3)

            wait_gather(p)

            @pl.when(i >= 1)
            def _():
                wait_write(p)

            combine(t, p)

        return carry

    lax.fori_loop(0, NCH // 3, triple, 0)
    wait_write(0)
    wait_write(1)
    wait_write(2)

    start = NCH * CB
    ca = pltpu.async_copy(ps_hbm.at[rowb.at[pl.ds(start, CB)]], abufs[0],
                          gsems[0])
    cb = pltpu.async_copy(pd_hbm.at[colb.at[pl.ds(start, CB)]], bbufs[0],
                          gsems[0])
    ca.wait()
    cb.wait()

    def addrow(i, carry):
        for k in range(4):
            cbufs[0][i, pl.ds(k * 16, 16)] = (
                abufs[0][i, pl.ds(k * 16, 16)]
                + bbufs[0][i, pl.ds(k * 16, 16)])
        return carry

    lax.fori_loop(0, REM, addrow, 0)
    pltpu.sync_copy(cbufs[0].at[pl.ds(0, REM)],
                    g_hbm.at[pl.ds(base + start, REM)])


def _sc_gather(ps, pd, row, col):
    run = pl.kernel(
        _sc_gather_body,
        out_type=jax.ShapeDtypeStruct((E, H), F32),
        mesh=plsc.VectorSubcoreMesh(core_axis_name="c", subcore_axis_name="s"),
        scratch_types=[
            pltpu.VMEM((PAD,), jnp.int32),
            pltpu.VMEM((PAD,), jnp.int32),
            pltpu.VMEM((CB, H), F32),
            pltpu.VMEM((CB, H), F32),
            pltpu.VMEM((CB, H), F32),
            pltpu.VMEM((CB, H), F32),
            pltpu.VMEM((CB, H), F32),
            pltpu.VMEM((CB, H), F32),
            pltpu.VMEM((CB, H), F32),
            pltpu.VMEM((CB, H), F32),
            pltpu.VMEM((CB, H), F32),
            pltpu.SemaphoreType.DMA,
            pltpu.SemaphoreType.DMA,
            pltpu.SemaphoreType.DMA,
            pltpu.SemaphoreType.DMA,
            pltpu.SemaphoreType.DMA,
            pltpu.SemaphoreType.DMA,
        ],
        compiler_params=pltpu.CompilerParams(use_tc_tiling_on_sc=False),
    )
    return run(ps, pd, row, col)


_NH = N // 2


_CBS = 128
_NCHUNKS_S = E // _CBS
_TRIPS = 80


def _sc_scatter_body(eo_hbm, col_hbm, out_hbm, colb0, colb1, ebuf0, ebuf1,
                     accum1d, sem0, sem1):
    cid = lax.axis_index("c")
    sid = lax.axis_index("s")
    wid = sid * NC + cid
    colbs, ebufs, sems = (colb0, colb1), (ebuf0, ebuf1), (sem0, sem1)

    def chunk_of(t):
        return jnp.minimum(wid + NW * t, _NCHUNKS_S - 1)

    def issue(t, p):
        c = chunk_of(t)
        pltpu.async_copy(col_hbm.at[pl.ds(c * _CBS, _CBS)],
                         colbs[p].at[pl.ds(0, _CBS)], sems[p])
        pltpu.async_copy(eo_hbm.at[pl.ds(c * _CBS, _CBS)], ebufs[p], sems[p])

    def wait(p):
        pltpu.make_async_copy(col_hbm.at[pl.ds(0, _CBS)],
                              colbs[p].at[pl.ds(0, _CBS)], sems[p]).wait()
        pltpu.make_async_copy(eo_hbm.at[pl.ds(0, _CBS)], ebufs[p],
                              sems[p]).wait()

    for h in range(2):
        node_base = h * _NH

        def zrow(i, carry):
            accum1d[pl.ds(i * 16, 16)] = _z16f()
            return carry

        lax.fori_loop(0, _NH + 1, zrow, 0)

        issue(0, 0)

        def pair(i, carry):
            for p in range(2):
                t = 2 * i + p

                @pl.when(t + 1 < _TRIPS)
                def _():
                    issue(t + 1, 1 - p)

                wait(p)

                @pl.when(wid + NW * t < _NCHUNKS_S)
                def _():
                    colb, ebuf = colbs[p], ebufs[p]

                    def group(g, carry2):
                        relv = colb[pl.ds(g * 16, 16)] - node_base
                        okv = jnp.logical_and(relv >= 0, relv < _NH)
                        offv = jnp.where(okv, relv, _NH) * DE
                        for l in range(16):
                            s16 = pl.ds(offv[l], 16)
                            accum1d[s16] = (accum1d[s16]
                                            + ebuf[g * 16 + l, pl.ds(0, DE)])
                        return carry2

                    lax.fori_loop(0, _CBS // 16, group, 0)

            return carry

        lax.fori_loop(0, _TRIPS // 2, pair, 0)
        pltpu.sync_copy(accum1d.at[pl.ds(0, _NH * DE)], out_hbm.at[wid, h])


def _sc_scatter(e_out, col):
    run = pl.kernel(
        _sc_scatter_body,
        out_type=jax.ShapeDtypeStruct((NW, 2, _NH * DE), F32),
        mesh=plsc.VectorSubcoreMesh(core_axis_name="c", subcore_axis_name="s"),
        scratch_types=[
            pltpu.VMEM((_CBS + 16,), jnp.int32),
            pltpu.VMEM((_CBS + 16,), jnp.int32),
            pltpu.VMEM((_CBS, DE), F32),
            pltpu.VMEM((_CBS, DE), F32),
            pltpu.VMEM(((_NH + 1) * DE,), F32),
            pltpu.SemaphoreType.DMA,
            pltpu.SemaphoreType.DMA,
        ],
    )
    return run(e_out, col)


def _tc_precompute(x, w_cat):
    def body(x_ref, w_ref, ps_ref, pd_ref):
        p = jnp.dot(x_ref[...], w_ref[...], preferred_element_type=F32)
        ps_ref[...] = p[:, :H]
        pd_ref[...] = p[:, H:]

    nb = 10
    rb = N // nb
    return pl.pallas_call(
        body,
        grid=(nb,),
        in_specs=[
            pl.BlockSpec((rb, D), lambda i: (i, 0)),
            pl.BlockSpec((D, 2 * H), lambda i: (0, 0)),
        ],
        out_specs=[
            pl.BlockSpec((rb, H), lambda i: (i, 0)),
            pl.BlockSpec((rb, H), lambda i: (i, 0)),
        ],
        out_shape=[
            jax.ShapeDtypeStruct((N, H), F32),
            jax.ShapeDtypeStruct((N, H), F32),
        ],
    )(x, w_cat)


def _tc_edge_mlp(g, edge_attr, w1e, b1, w2, b2):
    def body(g_ref, ea_ref, w1_ref, b1_ref, w2_ref, b2_ref, eo_ref):
        ea = ea_ref[...]
        h = jnp.maximum(
            g_ref[...] + jnp.dot(ea, w1_ref[...], preferred_element_type=F32)
            + b1_ref[...], 0.0)
        eo_ref[...] = (jnp.dot(h, w2_ref[...], preferred_element_type=F32)
                       + b2_ref[...] + ea)

    nb = 80
    rb = E // nb
    return pl.pallas_call(
        body,
        grid=(nb,),
        in_specs=[
            pl.BlockSpec((rb, H), lambda i: (i, 0)),
            pl.BlockSpec((rb, DE), lambda i: (i, 0)),
            pl.BlockSpec((DE, H), lambda i: (0, 0)),
            pl.BlockSpec((1, H), lambda i: (0, 0)),
            pl.BlockSpec((H, DE), lambda i: (0, 0)),
            pl.BlockSpec((1, DE), lambda i: (0, 0)),
        ],
        out_specs=pl.BlockSpec((rb, DE), lambda i: (i, 0)),
        out_shape=jax.ShapeDtypeStruct((E, DE), F32),
    )(g, edge_attr, w1e, b1, w2, b2)


def _tc_node_mlp(x, parts, u, wn_x, wn_a, wn_u, bn1, wn2, bn2):
    def body(x_ref, p_ref, u_ref, wx_ref, wa_ref, wu_ref, b1_ref, w2_ref,
             b2_ref, xo_ref):
        xv = x_ref[...]
        agg = jnp.sum(p_ref[...], axis=0)
        pre = (jnp.dot(xv, wx_ref[...], preferred_element_type=F32)
               + jnp.dot(agg, wa_ref[...], preferred_element_type=F32)
               + jnp.dot(u_ref[...], wu_ref[...], preferred_element_type=F32)
               + b1_ref[...])
        hn = jnp.maximum(pre, 0.0)
        xo_ref[...] = (jnp.dot(hn, w2_ref[...], preferred_element_type=F32)
                       + b2_ref[...] + xv)

    nb = 10
    rb = N // nb
    return pl.pallas_call(
        body,
        grid=(nb,),
        in_specs=[
            pl.BlockSpec((rb, D), lambda i: (i, 0)),
            pl.BlockSpec((NW, rb, DE), lambda i: (0, i, 0)),
            pl.BlockSpec((1, DG), lambda i: (0, 0)),
            pl.BlockSpec((D, H), lambda i: (0, 0)),
            pl.BlockSpec((DE, H), lambda i: (0, 0)),
            pl.BlockSpec((DG, H), lambda i: (0, 0)),
            pl.BlockSpec((1, H), lambda i: (0, 0)),
            pl.BlockSpec((H, D), lambda i: (0, 0)),
            pl.BlockSpec((1, D), lambda i: (0, 0)),
        ],
        out_specs=pl.BlockSpec((rb, D), lambda i: (i, 0)),
        out_shape=jax.ShapeDtypeStruct((N, D), F32),
    )(x, parts, u, wn_x, wn_a, wn_u, bn1, wn2, bn2)


DG = 16


def kernel(x, edge_index, edge_attr, u, batch, We1, be1, We2, be2,
           Wn1, bn1, Wn2, bn2):
    w1e = We1[2 * D:, :]
    wn_x = Wn1[:D, :]
    wn_a = Wn1[D:D + DE, :]
    wn_u = Wn1[D + DE:, :]

    w_cat = jnp.concatenate([We1[:D, :], We1[D:2 * D, :]], axis=1)

    row = edge_index[0]
    col = edge_index[1]
    ps, pd = _tc_precompute(x, w_cat)
    g = _sc_gather(ps, pd, row, col)
    e_out = _tc_edge_mlp(g, edge_attr, w1e, be1.reshape(1, H),
                         We2, be2.reshape(1, DE))
    parts = _sc_scatter(e_out, col).reshape(NW, N, DE)
    x_out = _tc_node_mlp(x, parts, u, wn_x, wn_a, wn_u,
                         bn1.reshape(1, H), Wn2, bn2.reshape(1, D))
    return (x_out, e_out, u)
